# baseline (speedup 1.0000x reference)
import functools

import jax
import jax.numpy as jnp
from jax import lax
from jax.experimental import pallas as pl
from jax.experimental.pallas import tpu as pltpu
from jax.experimental.pallas import tpu_sc as plsc

_N, _D = 16384, 64
_FLAT = _N * _D
_TBL = 1_000_000 * _D

_NC, _NS, _L = 2, 16, 16
_NW = _NC * _NS

_CH = 128
_ROWS = _FLAT // _CH
_NR = _ROWS // _NW

def _gather_body(t_hbm, c_hbm, out_hbm, ibuf, gbuf, sem):
    wid = lax.axis_index("s") * _NC + lax.axis_index("c")
    row0 = wid * _NR
    pltpu.sync_copy(c_hbm.at[pl.ds(row0, _NR), :], ibuf)

    lane = jnp.arange(_L, dtype=jnp.int32)
    offs = [lane + jnp.int32((m * _L) % _D) for m in range(_CH // _L)]

    def fire(r, carry):
        for m in range(_CH // _L):
            sl = pl.ds(m * _L, _L)
            ibuf[r, sl] = ibuf[r, sl] * jnp.int32(_D) + offs[m]
        pltpu.async_copy(t_hbm.at[ibuf.at[r]], gbuf.at[r], sem)
        return carry

    lax.fori_loop(0, _NR, fire, 0)
    pltpu.make_async_copy(out_hbm.at[pl.ds(row0, _NR), :], gbuf, sem).wait()
    pltpu.sync_copy(gbuf, out_hbm.at[pl.ds(row0, _NR), :])


@functools.cache
def _gather_kernel():
    mesh = plsc.VectorSubcoreMesh(
        core_axis_name="c", subcore_axis_name="s", num_cores=_NC, num_subcores=_NS
    )
    return pl.kernel(
        _gather_body,
        mesh=mesh,
        out_type=jax.ShapeDtypeStruct((_ROWS, _CH), jnp.float32),
        scratch_types=[
            pltpu.VMEM((_NR, _CH), jnp.int32),
            pltpu.VMEM((_NR, _CH), jnp.float32),
            pltpu.SemaphoreType.DMA,
        ],
    )


def kernel(t, d, c):
    idx = c + jnp.asarray(d, dtype=c.dtype)
    out = _gather_kernel()(t.reshape(_TBL), idx.reshape(_ROWS, _CH))
    return out.reshape(_N, _D)
